# baseline (speedup 1.0000x reference)
import jax
import jax.numpy as jnp
from jax.experimental import pallas as pl
from jax.experimental.pallas import tpu as pltpu

_N = 4096
_IN = 512
_H = 512
_L = 2
_BLK = 256
_NB = _N // _BLK
_K = 3


def _adj_cp(adj_hbm, abufs, sems, b):
    return pltpu.make_async_copy(
        adj_hbm.at[:, pl.ds(b * _BLK, _BLK), :], abufs[b % _K], sems[b % _K])


def _body(wg_ref, ww_ref, wy_ref, bg_ref, by_ref,
          f_hbm, adj_hbm, final_hbm, layers_hbm,
          fbuf, seq_s, a0, a1, a2, of0, of1, ol0, ol1,
          fsem, as0, as1, as2, ofs0, ofs1, ols0, ols1):
    abufs = (a0, a1, a2)
    asems = (as0, as1, as2)
    ofb, ofs = (of0, of1), (ofs0, ofs1)
    olb, ols = (ol0, ol1), (ols0, ols1)

    fcp = pltpu.make_async_copy(f_hbm, fbuf, fsem)
    fcp.start()
    for k in range(_K):
        _adj_cp(adj_hbm, abufs, asems, k).start()

    fcp.wait()
    f16 = fbuf[...]
    vs = []
    for l in range(_L):
        wg = wg_ref[l]
        seq_s[l] = jax.lax.dot_general(
            f16, wg, (((1,), (1,)), ((), ())),
            preferred_element_type=jnp.float32)
        vs.append(jnp.sum(ww_ref[l] * wy_ref[l, 0][:, None], axis=0))

    for b in range(_NB):
        k = b % _K
        _adj_cp(adj_hbm, abufs, asems, b).wait()
        embs = []
        for l in range(_L):
            a = abufs[k][l]
            e = jax.lax.dot_general(
                a, seq_s[l], (((1,), (0,)), ((), ())),
                preferred_element_type=jnp.float32)
            embs.append(jnp.maximum(e + bg_ref[l, 0], 0.0))
        if b + _K < _NB:
            _adj_cp(adj_hbm, abufs, asems, b + _K).start()
        ws = []
        for l in range(_L):
            s = jnp.sum(embs[l] * vs[l], axis=1, keepdims=True) + by_ref[0, l]
            ws.append(jnp.exp(jnp.tanh(s)))
        inv = 1.0 / (ws[0] + ws[1])
        s2 = b % 2
        if b >= 2:
            pltpu.make_async_copy(
                ofb[s2], final_hbm.at[pl.ds((b - 2) * _BLK, _BLK), :],
                ofs[s2]).wait()
            pltpu.make_async_copy(
                olb[s2], layers_hbm.at[:, pl.ds((b - 2) * _BLK, _BLK), :],
                ols[s2]).wait()
        ofb[s2][...] = (ws[0] * embs[0] + ws[1] * embs[1]) * inv
        for l in range(_L):
            olb[s2][l] = embs[l]
        pltpu.make_async_copy(
            ofb[s2], final_hbm.at[pl.ds(b * _BLK, _BLK), :], ofs[s2]).start()
        pltpu.make_async_copy(
            olb[s2], layers_hbm.at[:, pl.ds(b * _BLK, _BLK), :], ols[s2]).start()

    for b in (_NB - 2, _NB - 1):
        s2 = b % 2
        pltpu.make_async_copy(
            ofb[s2], final_hbm.at[pl.ds(b * _BLK, _BLK), :], ofs[s2]).wait()
        pltpu.make_async_copy(
            olb[s2], layers_hbm.at[:, pl.ds(b * _BLK, _BLK), :], ols[s2]).wait()


def kernel(features, adj_list, W_gcn, b_gcn, W_w, W_y, b_y, sparse):
    f = features[0]
    adj = adj_list[:, 0]
    wy3 = W_y.reshape(_L, 1, _H)
    bg3 = b_gcn.reshape(_L, 1, _H)
    by2 = b_y.reshape(1, _L)

    vmem = pl.BlockSpec(memory_space=pltpu.MemorySpace.VMEM)
    hbm = pl.BlockSpec(memory_space=pltpu.MemorySpace.HBM)
    final, layers = pl.pallas_call(
        _body,
        in_specs=[vmem, vmem, vmem, vmem, vmem, hbm, hbm],
        out_specs=[hbm, hbm],
        out_shape=[
            jax.ShapeDtypeStruct((_N, _H), jnp.float32),
            jax.ShapeDtypeStruct((_L, _N, _H), jnp.float32),
        ],
        scratch_shapes=[
            pltpu.VMEM((_N, _IN), jnp.float32),
            pltpu.VMEM((_L, _N, _H), jnp.float32),
            pltpu.VMEM((_L, _BLK, _N), jnp.float32),
            pltpu.VMEM((_L, _BLK, _N), jnp.float32),
            pltpu.VMEM((_L, _BLK, _N), jnp.float32),
            pltpu.VMEM((_BLK, _H), jnp.float32),
            pltpu.VMEM((_BLK, _H), jnp.float32),
            pltpu.VMEM((_L, _BLK, _H), jnp.float32),
            pltpu.VMEM((_L, _BLK, _H), jnp.float32),
            pltpu.SemaphoreType.DMA,
            pltpu.SemaphoreType.DMA,
            pltpu.SemaphoreType.DMA,
            pltpu.SemaphoreType.DMA,
            pltpu.SemaphoreType.DMA,
            pltpu.SemaphoreType.DMA,
            pltpu.SemaphoreType.DMA,
            pltpu.SemaphoreType.DMA,
        ],
    )(W_gcn, W_w, wy3, bg3, by2, f, adj)

    return (final, layers)
